# baseline (speedup 1.0000x reference)
import functools

import jax
import jax.numpy as jnp
from jax import lax
from jax.experimental import pallas as pl
from jax.experimental.pallas import tpu as pltpu
from jax.experimental.pallas import tpu_sc as plsc

B, N, F = 8, 50000, 128

N_SC = 22000
N_TC = N - N_SC

QUARTERS = 4
ROWS_W = N_SC // QUARTERS
RCHUNK = 250
NCHUNK = ROWS_W // RCHUNK
CELEMS = RCHUNK * F
NVREG = F // 16

TCHUNK = 2000
TC_OFF = N_SC // TCHUNK
NTCHUNK = N_TC // TCHUNK


def _sc_body(x_hbm, out_hbm, buf, stage, cbuf, shared, sem0, sem1):
    c = lax.axis_index("c")
    s = lax.axis_index("s")
    wid = c * 16 + s
    b = wid // QUARTERS
    q = wid % QUARTERS
    base = b * (N * F) + q * (ROWS_W * F)
    sems = (sem0, sem1)

    def src(t):
        return x_hbm.at[pl.ds(base + t * CELEMS, CELEMS)]

    pltpu.async_copy(src(0), buf.at[0], sem0)
    pltpu.async_copy(src(1), buf.at[1], sem1)

    def chunk_body(g, accs):
        for slot in range(2):
            t = g * 2 + slot
            pltpu.make_async_copy(src(t), buf.at[slot], sems[slot]).wait()
            bslot = buf.at[slot]

            def row_body(r, a, bslot=bslot):
                ro = r * F
                return tuple(
                    a[k] + bslot[pl.ds(ro + k * 16, 16)]
                    for k in range(NVREG)
                )

            accs = lax.fori_loop(0, RCHUNK, row_body, accs, unroll=4)

            @pl.when(t + 2 < NCHUNK)
            def _prefetch():
                pltpu.async_copy(src(t + 2), buf.at[slot], sems[slot])

        return accs

    zero = jnp.zeros((16,), jnp.float32)
    accs = lax.fori_loop(0, NCHUNK // 2, chunk_body, (zero,) * NVREG)

    for k in range(NVREG):
        stage[pl.ds(k * 16, 16)] = accs[k]
    pltpu.sync_copy(stage, shared.at[s])
    plsc.subcore_barrier()

    @pl.when(q == 0)
    def _combine():
        pltpu.sync_copy(shared.at[pl.ds(s, QUARTERS)], cbuf)
        for k in range(NVREG):
            tot = (
                cbuf[0, pl.ds(k * 16, 16)]
                + cbuf[1, pl.ds(k * 16, 16)]
                + cbuf[2, pl.ds(k * 16, 16)]
                + cbuf[3, pl.ds(k * 16, 16)]
            )
            stage[pl.ds(k * 16, 16)] = tot
        pltpu.sync_copy(stage, out_hbm.at[b])


def _sc_partial(x):
    mesh = plsc.VectorSubcoreMesh(core_axis_name="c", subcore_axis_name="s")
    sc = pl.kernel(
        _sc_body,
        mesh=mesh,
        out_type=jax.ShapeDtypeStruct((B, F), jnp.float32),
        scratch_types=[
            pltpu.VMEM((2, CELEMS), jnp.float32),
            pltpu.VMEM((F,), jnp.float32),
            pltpu.VMEM((QUARTERS, F), jnp.float32),
            pltpu.VMEM_SHARED((16, F), jnp.float32),
            pltpu.SemaphoreType.DMA,
            pltpu.SemaphoreType.DMA,
        ],
    )
    return sc(x.reshape(-1))


def _tc_body(x_ref, o_ref):
    j = pl.program_id(0)

    @pl.when(j == 0)
    def _init():
        o_ref[...] = jnp.zeros_like(o_ref)

    o_ref[...] += jnp.sum(x_ref[...], axis=1)


def _tc_partial(x):
    return pl.pallas_call(
        _tc_body,
        grid=(NTCHUNK,),
        in_specs=[pl.BlockSpec((B, TCHUNK, F), lambda j: (0, j + TC_OFF, 0))],
        out_specs=pl.BlockSpec((B, F), lambda j: (0, 0)),
        out_shape=jax.ShapeDtypeStruct((B, F), jnp.float32),
    )(x)


def _combine_body(a_ref, b_ref, o_ref):
    o_ref[...] = (a_ref[...] + b_ref[...]) * (1.0 / N)


def _combine(a, b):
    return pl.pallas_call(
        _combine_body,
        out_shape=jax.ShapeDtypeStruct((B, F), jnp.float32),
    )(a, b)


@jax.jit
def kernel(x):
    sc_part = _sc_partial(x)
    tc_part = _tc_partial(x)
    return _combine(sc_part, tc_part)


# speedup vs baseline: 1.0182x; 1.0182x over previous
import functools

import jax
import jax.numpy as jnp
from jax import lax
from jax.experimental import pallas as pl
from jax.experimental.pallas import tpu as pltpu
from jax.experimental.pallas import tpu_sc as plsc

B, N, F = 8, 50000, 128

N_SC = 12000
N_TC = N - N_SC

QUARTERS = 4
ROWS_W = N_SC // QUARTERS
RCHUNK = 250
NCHUNK = ROWS_W // RCHUNK
CELEMS = RCHUNK * F
NVREG = F // 16

TCHUNK = 2000
TC_OFF = N_SC // TCHUNK
NTCHUNK = N_TC // TCHUNK


def _sc_body(x_hbm, out_hbm, buf, stage, cbuf, shared, sem0, sem1):
    c = lax.axis_index("c")
    s = lax.axis_index("s")
    wid = c * 16 + s
    b = wid // QUARTERS
    q = wid % QUARTERS
    base = b * (N * F) + q * (ROWS_W * F)
    sems = (sem0, sem1)

    def src(t):
        return x_hbm.at[pl.ds(base + t * CELEMS, CELEMS)]

    pltpu.async_copy(src(0), buf.at[0], sem0)
    pltpu.async_copy(src(1), buf.at[1], sem1)

    def chunk_body(g, accs):
        for slot in range(2):
            t = g * 2 + slot
            pltpu.make_async_copy(src(t), buf.at[slot], sems[slot]).wait()
            bslot = buf.at[slot]

            def row_body(r, a, bslot=bslot):
                ro = r * F
                return tuple(
                    a[k] + bslot[pl.ds(ro + k * 16, 16)]
                    for k in range(NVREG)
                )

            accs = lax.fori_loop(0, RCHUNK, row_body, accs, unroll=4)

            @pl.when(t + 2 < NCHUNK)
            def _prefetch():
                pltpu.async_copy(src(t + 2), buf.at[slot], sems[slot])

        return accs

    zero = jnp.zeros((16,), jnp.float32)
    accs = lax.fori_loop(0, NCHUNK // 2, chunk_body, (zero,) * NVREG)

    for k in range(NVREG):
        stage[pl.ds(k * 16, 16)] = accs[k]
    pltpu.sync_copy(stage, shared.at[s])
    plsc.subcore_barrier()

    @pl.when(q == 0)
    def _combine():
        pltpu.sync_copy(shared.at[pl.ds(s, QUARTERS)], cbuf)
        for k in range(NVREG):
            tot = (
                cbuf[0, pl.ds(k * 16, 16)]
                + cbuf[1, pl.ds(k * 16, 16)]
                + cbuf[2, pl.ds(k * 16, 16)]
                + cbuf[3, pl.ds(k * 16, 16)]
            )
            stage[pl.ds(k * 16, 16)] = tot
        pltpu.sync_copy(stage, out_hbm.at[b])


def _sc_partial(x):
    mesh = plsc.VectorSubcoreMesh(core_axis_name="c", subcore_axis_name="s")
    sc = pl.kernel(
        _sc_body,
        mesh=mesh,
        out_type=jax.ShapeDtypeStruct((B, F), jnp.float32),
        scratch_types=[
            pltpu.VMEM((2, CELEMS), jnp.float32),
            pltpu.VMEM((F,), jnp.float32),
            pltpu.VMEM((QUARTERS, F), jnp.float32),
            pltpu.VMEM_SHARED((16, F), jnp.float32),
            pltpu.SemaphoreType.DMA,
            pltpu.SemaphoreType.DMA,
        ],
    )
    return sc(x.reshape(-1))


def _tc_body(x_ref, o_ref):
    j = pl.program_id(0)

    @pl.when(j == 0)
    def _init():
        o_ref[...] = jnp.zeros_like(o_ref)

    o_ref[...] += jnp.sum(x_ref[...], axis=1)


def _tc_partial(x):
    return pl.pallas_call(
        _tc_body,
        grid=(NTCHUNK,),
        in_specs=[pl.BlockSpec((B, TCHUNK, F), lambda j: (0, j + TC_OFF, 0))],
        out_specs=pl.BlockSpec((B, F), lambda j: (0, 0)),
        out_shape=jax.ShapeDtypeStruct((B, F), jnp.float32),
    )(x)


def _combine_body(a_ref, b_ref, o_ref):
    o_ref[...] = (a_ref[...] + b_ref[...]) * (1.0 / N)


def _combine(a, b):
    return pl.pallas_call(
        _combine_body,
        out_shape=jax.ShapeDtypeStruct((B, F), jnp.float32),
    )(a, b)


@jax.jit
def kernel(x):
    sc_part = _sc_partial(x)
    tc_part = _tc_partial(x)
    return _combine(sc_part, tc_part)


# speedup vs baseline: 1.0212x; 1.0030x over previous
import functools

import jax
import jax.numpy as jnp
from jax import lax
from jax.experimental import pallas as pl
from jax.experimental.pallas import tpu as pltpu
from jax.experimental.pallas import tpu_sc as plsc

B, N, F = 8, 50000, 128

N_SC = 10000
N_TC = N - N_SC

QUARTERS = 4
ROWS_W = N_SC // QUARTERS
RCHUNK = 250
NCHUNK = ROWS_W // RCHUNK
CELEMS = RCHUNK * F
NVREG = F // 16

TCHUNK = 2000
TC_OFF = N_SC // TCHUNK
NTCHUNK = N_TC // TCHUNK


def _sc_body(x_hbm, out_hbm, buf, stage, cbuf, shared, sem0, sem1):
    c = lax.axis_index("c")
    s = lax.axis_index("s")
    wid = c * 16 + s
    b = wid // QUARTERS
    q = wid % QUARTERS
    base = b * (N * F) + q * (ROWS_W * F)
    sems = (sem0, sem1)

    def src(t):
        return x_hbm.at[pl.ds(base + t * CELEMS, CELEMS)]

    pltpu.async_copy(src(0), buf.at[0], sem0)
    pltpu.async_copy(src(1), buf.at[1], sem1)

    def chunk_body(g, accs):
        for slot in range(2):
            t = g * 2 + slot
            pltpu.make_async_copy(src(t), buf.at[slot], sems[slot]).wait()
            bslot = buf.at[slot]

            def row_body(r, a, bslot=bslot):
                ro = r * F
                return tuple(
                    a[k] + bslot[pl.ds(ro + k * 16, 16)]
                    for k in range(NVREG)
                )

            accs = lax.fori_loop(0, RCHUNK, row_body, accs, unroll=4)

            @pl.when(t + 2 < NCHUNK)
            def _prefetch():
                pltpu.async_copy(src(t + 2), buf.at[slot], sems[slot])

        return accs

    zero = jnp.zeros((16,), jnp.float32)
    accs = lax.fori_loop(0, NCHUNK // 2, chunk_body, (zero,) * NVREG)

    for k in range(NVREG):
        stage[pl.ds(k * 16, 16)] = accs[k]
    pltpu.sync_copy(stage, shared.at[s])
    plsc.subcore_barrier()

    @pl.when(q == 0)
    def _combine():
        pltpu.sync_copy(shared.at[pl.ds(s, QUARTERS)], cbuf)
        for k in range(NVREG):
            tot = (
                cbuf[0, pl.ds(k * 16, 16)]
                + cbuf[1, pl.ds(k * 16, 16)]
                + cbuf[2, pl.ds(k * 16, 16)]
                + cbuf[3, pl.ds(k * 16, 16)]
            )
            stage[pl.ds(k * 16, 16)] = tot
        pltpu.sync_copy(stage, out_hbm.at[b])


def _sc_partial(x):
    mesh = plsc.VectorSubcoreMesh(core_axis_name="c", subcore_axis_name="s")
    sc = pl.kernel(
        _sc_body,
        mesh=mesh,
        out_type=jax.ShapeDtypeStruct((B, F), jnp.float32),
        scratch_types=[
            pltpu.VMEM((2, CELEMS), jnp.float32),
            pltpu.VMEM((F,), jnp.float32),
            pltpu.VMEM((QUARTERS, F), jnp.float32),
            pltpu.VMEM_SHARED((16, F), jnp.float32),
            pltpu.SemaphoreType.DMA,
            pltpu.SemaphoreType.DMA,
        ],
    )
    return sc(x.reshape(-1))


def _tc_body(x_ref, o_ref):
    j = pl.program_id(0)

    @pl.when(j == 0)
    def _init():
        o_ref[...] = jnp.zeros_like(o_ref)

    o_ref[...] += jnp.sum(x_ref[...], axis=1)


def _tc_partial(x):
    return pl.pallas_call(
        _tc_body,
        grid=(NTCHUNK,),
        in_specs=[pl.BlockSpec((B, TCHUNK, F), lambda j: (0, j + TC_OFF, 0))],
        out_specs=pl.BlockSpec((B, F), lambda j: (0, 0)),
        out_shape=jax.ShapeDtypeStruct((B, F), jnp.float32),
    )(x)


def _combine_body(a_ref, b_ref, o_ref):
    o_ref[...] = (a_ref[...] + b_ref[...]) * (1.0 / N)


def _combine(a, b):
    return pl.pallas_call(
        _combine_body,
        out_shape=jax.ShapeDtypeStruct((B, F), jnp.float32),
    )(a, b)


@jax.jit
def kernel(x):
    sc_part = _sc_partial(x)
    tc_part = _tc_partial(x)
    return _combine(sc_part, tc_part)
